# Initial kernel scaffold; baseline (speedup 1.0000x reference)
import jax
import jax.numpy as jnp
from jax.experimental import pallas as pl


def kernel(x, edge_index, W_in, b_in, Wb1, bb1, Wb2, bb2, W_out, b_out):
    raise NotImplementedError("write your pallas kernel here")


# speedup vs baseline: 3.0760x; 3.0760x over previous
import functools

import jax
import jax.numpy as jnp
from jax import lax
from jax.experimental import pallas as pl
from jax.experimental.pallas import tpu as pltpu, tpu_sc as plsc

N = 10000
E = 160000
IN_DIM = 256
HID = 512
OUT = 3
NB = 3

N_PAD = 10240
E_PAD = 163840

NC = 2
NS = 16
BLK = 128


def _make_agg(C, D):
    rows = N_PAD // NS
    nblk = E_PAD // (NS * BLK)

    mesh = plsc.VectorSubcoreMesh(core_axis_name="c", subcore_axis_name="s")
    out_type = tuple(
        jax.ShapeDtypeStruct((N_PAD, D), jnp.float32) for _ in range(C))
    scratch = [
        pltpu.VMEM_SHARED((N_PAD, D), jnp.float32),
        pltpu.VMEM((BLK,), jnp.int32),
        pltpu.VMEM((BLK,), jnp.int32),
        pltpu.VMEM((BLK, D), jnp.float32),
        pltpu.SemaphoreType.DMA,
    ]

    @functools.partial(pl.kernel, out_type=out_type, mesh=mesh,
                       scratch_types=scratch)
    def agg(*refs):
        h_refs = refs[:C]
        src_ref, dst_ref = refs[C], refs[C + 1]
        out_refs = refs[C + 2:2 * C + 2]
        acc, src_v, dst_v, msg_v, sem = refs[2 * C + 2:]

        cid = lax.axis_index("c")
        sid = lax.axis_index("s")
        r0 = sid * rows

        for ch in range(C):
            @pl.when(cid == ch % NC)
            def _(ch=ch):
                pltpu.sync_copy(h_refs[ch].at[pl.ds(r0, rows)],
                                acc.at[pl.ds(r0, rows)])
                plsc.subcore_barrier()

                def body(i, carry):
                    off = (sid * nblk + i) * BLK
                    pltpu.sync_copy(src_ref.at[pl.ds(off, BLK)], src_v)
                    pltpu.sync_copy(dst_ref.at[pl.ds(off, BLK)], dst_v)
                    pltpu.async_copy(h_refs[ch].at[src_v], msg_v, sem).wait()
                    pltpu.sync_copy(msg_v, acc.at[dst_v], add=True)
                    return carry

                lax.fori_loop(0, nblk, body, 0)
                plsc.subcore_barrier()
                pltpu.sync_copy(acc.at[pl.ds(r0, rows)],
                                out_refs[ch].at[pl.ds(r0, rows)])

    return agg


_BM = 512


def _mm(x_ch, w, dinv, cin, cout):
    def body(x_ref, w_ref, d_ref, o_ref):
        acc = jnp.zeros((_BM, 128), jnp.float32)
        for k in range(cin):
            acc += jnp.dot(x_ref[k], w_ref[k * 128:(k + 1) * 128, :],
                           preferred_element_type=jnp.float32)
        o_ref[0] = acc * d_ref[...]

    return pl.pallas_call(
        body,
        grid=(N_PAD // _BM, cout),
        in_specs=[
            pl.BlockSpec((cin, _BM, 128), lambda m, c: (0, m, 0)),
            pl.BlockSpec((cin * 128, 128), lambda m, c: (0, c)),
            pl.BlockSpec((_BM, 1), lambda m, c: (m, 0)),
        ],
        out_specs=pl.BlockSpec((1, _BM, 128), lambda m, c: (c, m, 0)),
        out_shape=jax.ShapeDtypeStruct((cout, N_PAD, 128), jnp.float32),
    )(x_ch, w, dinv)


def _fin(s_ch, dinv, b_ch, res_ch, C, relu):
    has_res = res_ch is not None

    def body(*refs):
        if has_res:
            s_ref, d_ref, b_ref, r_ref, o_ref = refs
        else:
            s_ref, d_ref, b_ref, o_ref = refs
        y = s_ref[0] * d_ref[...] + b_ref[0, 0][None, :]
        if has_res:
            y = y + r_ref[0]
        if relu:
            y = jnp.maximum(y, 0.0)
        o_ref[0] = y

    in_specs = [
        pl.BlockSpec((1, _BM, 128), lambda m, c: (c, m, 0)),
        pl.BlockSpec((_BM, 1), lambda m, c: (m, 0)),
        pl.BlockSpec((1, 1, 128), lambda m, c: (c, 0, 0)),
    ]
    args = [s_ch, dinv, b_ch]
    if has_res:
        in_specs.append(pl.BlockSpec((1, _BM, 128), lambda m, c: (c, m, 0)))
        args.append(res_ch)

    return pl.pallas_call(
        body,
        grid=(N_PAD // _BM, C),
        in_specs=in_specs,
        out_specs=pl.BlockSpec((1, _BM, 128), lambda m, c: (c, m, 0)),
        out_shape=jax.ShapeDtypeStruct((C, N_PAD, 128), jnp.float32),
    )(*args)


def _dinv_kernel(dacc):
    def body(a_ref, o_ref):
        deg = a_ref[:, 0:1]
        row = (pl.program_id(0) * _BM
               + lax.broadcasted_iota(jnp.int32, (_BM, 1), 0))
        o_ref[...] = jnp.where(row < N, lax.rsqrt(deg), 0.0)

    return pl.pallas_call(
        body,
        grid=(N_PAD // _BM,),
        in_specs=[pl.BlockSpec((_BM, 128), lambda m: (m, 0))],
        out_specs=pl.BlockSpec((_BM, 1), lambda m: (m, 0)),
        out_shape=jax.ShapeDtypeStruct((N_PAD, 1), jnp.float32),
    )(dacc)


def kernel(x, edge_index, W_in, b_in, Wb1, bb1, Wb2, bb2, W_out, b_out):
    pad = jnp.full((E_PAD - E,), N_PAD - 1, dtype=jnp.int32)
    src = jnp.concatenate([edge_index[0], pad])
    dst = jnp.concatenate([edge_index[1], pad])

    x_pad = jnp.zeros((N_PAD, IN_DIM), jnp.float32).at[:N].set(x)
    x_ch = x_pad.reshape(N_PAD, 2, 128).transpose(1, 0, 2)

    agg4 = _make_agg(4, 128)
    agg1_wide = _make_agg(1, 128)

    ones_tab = jnp.ones((N_PAD, 128), jnp.float32)
    (dacc,) = agg1_wide(ones_tab, src, dst)
    dinv = _dinv_kernel(dacc)

    def gcn(h_ch, cin, w, b, res_ch, relu):
        cout = w.shape[1] // 128
        hp = _mm(h_ch, w, dinv, cin, cout)
        s = agg4(hp[0], hp[1], hp[2], hp[3], src, dst) if cout == 4 else \
            agg1_wide(hp[0], src, dst)
        s = jnp.stack(s)
        return _fin(s, dinv, b.reshape(cout, 1, 128), res_ch, cout, relu)

    h = gcn(x_ch, 2, W_in, b_in, None, True)
    for i in range(NB):
        t = gcn(h, 4, Wb1[i], bb1[i], None, True)
        h = gcn(t, 4, Wb2[i], bb2[i], h, True)

    W_out_pad = jnp.zeros((HID, 128), jnp.float32).at[:, :OUT].set(W_out)
    b_out_pad = jnp.zeros((128,), jnp.float32).at[:OUT].set(b_out)
    y = gcn(h, 4, W_out_pad, b_out_pad, None, False)
    return y[0, :N, :OUT]


# speedup vs baseline: 4.2057x; 1.3673x over previous
import functools

import jax
import jax.numpy as jnp
from jax import lax
from jax.experimental import pallas as pl
from jax.experimental.pallas import tpu as pltpu, tpu_sc as plsc

N = 10000
E = 160000
IN_DIM = 256
HID = 512
OUT = 3
NB = 3

N_PAD = 10240
E_PAD = 163840

NC = 2
NS = 16
BLK = 128


def _make_agg(C, D, gather=True):
    rows = N_PAD // NS
    nblk = E_PAD // (NS * BLK)

    mesh = plsc.VectorSubcoreMesh(core_axis_name="c", subcore_axis_name="s")
    out_type = tuple(
        jax.ShapeDtypeStruct((N_PAD, D), jnp.float32) for _ in range(C))
    scratch = [
        pltpu.VMEM_SHARED((N_PAD, D), jnp.float32),
        pltpu.VMEM((2, BLK), jnp.int32),
        pltpu.VMEM((2, BLK), jnp.int32),
        pltpu.VMEM((BLK, D), jnp.float32),
        pltpu.VMEM((BLK, D), jnp.float32),
        pltpu.SemaphoreType.DMA,
        pltpu.SemaphoreType.DMA,
    ]

    @functools.partial(pl.kernel, out_type=out_type, mesh=mesh,
                       scratch_types=scratch)
    def agg(*refs):
        h_refs = refs[:C]
        edges_ref = refs[C]
        out_refs = refs[C + 1:2 * C + 1]
        acc, ib0, ib1, msg0, msg1, sem0, sem1 = refs[2 * C + 1:]

        cid = lax.axis_index("c")
        sid = lax.axis_index("s")
        r0 = sid * rows
        e0 = sid * nblk

        if not gather:
            pltpu.sync_copy(h_refs[0].at[pl.ds(0, BLK)], msg0)

        for ch in range(C):
            @pl.when(cid == ch % NC)
            def _(ch=ch):
                pltpu.sync_copy(h_refs[ch].at[pl.ds(r0, rows)],
                                acc.at[pl.ds(r0, rows)])
                plsc.subcore_barrier()

                if gather:
                    pltpu.sync_copy(edges_ref.at[e0], ib0)
                    pltpu.async_copy(h_refs[ch].at[ib0.at[0]], msg0, sem0)

                    def body(j, carry):
                        b0 = 2 * j
                        pltpu.sync_copy(edges_ref.at[e0 + b0 + 1], ib1)
                        pltpu.async_copy(h_refs[ch].at[ib1.at[0]], msg1, sem1)
                        pltpu.make_async_copy(
                            h_refs[ch].at[ib0.at[0]], msg0, sem0).wait()
                        pltpu.sync_copy(msg0, acc.at[ib0.at[1]], add=True)

                        @pl.when(b0 + 2 < nblk)
                        def _():
                            pltpu.sync_copy(edges_ref.at[e0 + b0 + 2], ib0)
                            pltpu.async_copy(h_refs[ch].at[ib0.at[0]],
                                             msg0, sem0)
                        pltpu.make_async_copy(
                            h_refs[ch].at[ib1.at[0]], msg1, sem1).wait()
                        pltpu.sync_copy(msg1, acc.at[ib1.at[1]], add=True)
                        return carry

                    lax.fori_loop(0, nblk // 2, body, 0)
                else:
                    def body(i, carry):
                        pltpu.sync_copy(edges_ref.at[e0 + i], ib0)
                        pltpu.sync_copy(msg0, acc.at[ib0.at[1]], add=True)
                        return carry

                    lax.fori_loop(0, nblk, body, 0)

                plsc.subcore_barrier()
                pltpu.sync_copy(acc.at[pl.ds(r0, rows)],
                                out_refs[ch].at[pl.ds(r0, rows)])

    return agg


_BM = 512


def _mm(x_ch, w, dinv, cin, cout):
    def body(x_ref, w_ref, d_ref, o_ref):
        acc = jnp.zeros((_BM, 128), jnp.float32)
        for k in range(cin):
            acc += jnp.dot(x_ref[k], w_ref[k * 128:(k + 1) * 128, :],
                           preferred_element_type=jnp.float32)
        o_ref[0] = acc * d_ref[...]

    return pl.pallas_call(
        body,
        grid=(N_PAD // _BM, cout),
        in_specs=[
            pl.BlockSpec((cin, _BM, 128), lambda m, c: (0, m, 0)),
            pl.BlockSpec((cin * 128, 128), lambda m, c: (0, c)),
            pl.BlockSpec((_BM, 1), lambda m, c: (m, 0)),
        ],
        out_specs=pl.BlockSpec((1, _BM, 128), lambda m, c: (c, m, 0)),
        out_shape=jax.ShapeDtypeStruct((cout, N_PAD, 128), jnp.float32),
    )(x_ch, w, dinv)


def _fin(s_ch, dinv, b_ch, res_ch, C, relu):
    has_res = res_ch is not None

    def body(*refs):
        if has_res:
            s_ref, d_ref, b_ref, r_ref, o_ref = refs
        else:
            s_ref, d_ref, b_ref, o_ref = refs
        y = s_ref[0] * d_ref[...] + b_ref[0, 0][None, :]
        if has_res:
            y = y + r_ref[0]
        if relu:
            y = jnp.maximum(y, 0.0)
        o_ref[0] = y

    in_specs = [
        pl.BlockSpec((1, _BM, 128), lambda m, c: (c, m, 0)),
        pl.BlockSpec((_BM, 1), lambda m, c: (m, 0)),
        pl.BlockSpec((1, 1, 128), lambda m, c: (c, 0, 0)),
    ]
    args = [s_ch, dinv, b_ch]
    if has_res:
        in_specs.append(pl.BlockSpec((1, _BM, 128), lambda m, c: (c, m, 0)))
        args.append(res_ch)

    return pl.pallas_call(
        body,
        grid=(N_PAD // _BM, C),
        in_specs=in_specs,
        out_specs=pl.BlockSpec((1, _BM, 128), lambda m, c: (c, m, 0)),
        out_shape=jax.ShapeDtypeStruct((C, N_PAD, 128), jnp.float32),
    )(*args)


def _dinv_kernel(dacc):
    def body(a_ref, o_ref):
        deg = a_ref[:, 0:1]
        row = (pl.program_id(0) * _BM
               + lax.broadcasted_iota(jnp.int32, (_BM, 1), 0))
        o_ref[...] = jnp.where(row < N, lax.rsqrt(deg), 0.0)

    return pl.pallas_call(
        body,
        grid=(N_PAD // _BM,),
        in_specs=[pl.BlockSpec((_BM, 128), lambda m: (m, 0))],
        out_specs=pl.BlockSpec((_BM, 1), lambda m: (m, 0)),
        out_shape=jax.ShapeDtypeStruct((N_PAD, 1), jnp.float32),
    )(dacc)


def kernel(x, edge_index, W_in, b_in, Wb1, bb1, Wb2, bb2, W_out, b_out):
    pad = jnp.full((E_PAD - E,), N_PAD - 1, dtype=jnp.int32)
    src = jnp.concatenate([edge_index[0], pad]).reshape(E_PAD // BLK, BLK)
    dst = jnp.concatenate([edge_index[1], pad]).reshape(E_PAD // BLK, BLK)
    edges = jnp.stack([src, dst], axis=1)

    x_pad = jnp.zeros((N_PAD, IN_DIM), jnp.float32).at[:N].set(x)
    x_ch = x_pad.reshape(N_PAD, 2, 128).transpose(1, 0, 2)

    agg4 = _make_agg(4, 128)
    agg1_wide = _make_agg(1, 128)
    agg_deg = _make_agg(1, 128, gather=False)

    ones_tab = jnp.ones((N_PAD, 128), jnp.float32)
    (dacc,) = agg_deg(ones_tab, edges)
    dinv = _dinv_kernel(dacc)

    def gcn(h_ch, cin, w, b, res_ch, relu):
        cout = w.shape[1] // 128
        hp = _mm(h_ch, w, dinv, cin, cout)
        s = agg4(hp[0], hp[1], hp[2], hp[3], edges) if cout == 4 else \
            agg1_wide(hp[0], edges)
        s = jnp.stack(s)
        return _fin(s, dinv, b.reshape(cout, 1, 128), res_ch, cout, relu)

    h = gcn(x_ch, 2, W_in, b_in, None, True)
    for i in range(NB):
        t = gcn(h, 4, Wb1[i], bb1[i], None, True)
        h = gcn(t, 4, Wb2[i], bb2[i], h, True)

    W_out_pad = jnp.zeros((HID, 128), jnp.float32).at[:, :OUT].set(W_out)
    b_out_pad = jnp.zeros((128,), jnp.float32).at[:OUT].set(b_out)
    y = gcn(h, 4, W_out_pad, b_out_pad, None, False)
    return y[0, :N, :OUT]
